# baseline (speedup 1.0000x reference)
import functools

import numpy as np
import jax
import jax.numpy as jnp
from jax.experimental import pallas as pl
from jax.experimental.pallas import tpu as pltpu

_D = 18
_GS = 9
_NC = 512
_NOUT = 2 * _NC + 3
_SQRT_D = np.float32(np.sqrt(np.float32(18.0)))
_QS = np.float32(np.float32(1.0) / _SQRT_D)
_HALF_QS = np.float32(_QS / np.float32(2.0))
_ROWS = 1024


def _weights():
    codes = np.arange(_NC)
    gb = 2 ** np.arange(_GS - 1, -1, -1)
    cb = (((codes[:, None] // gb) % 2) * 2 - 1).astype(np.float32)
    w = np.zeros((3 * _D, _NOUT), np.float32)
    w[:_GS, :_NC] = cb.T
    w[_GS:_D, _NC:2 * _NC] = cb.T
    w[_D:2 * _D] = w[:_D]
    basis = (2.0 ** np.arange(_D - 1, -1, -1)).astype(np.float32)
    w[2 * _D:, 2 * _NC] = basis
    w[2 * _D:2 * _D + _GS, 2 * _NC + 1] = basis[_GS:]
    w[2 * _D + _GS:, 2 * _NC + 2] = basis[_GS:]
    return jnp.asarray(w)


def _group_ones():
    m = np.zeros((2, 2 * _NC), np.float32)
    m[0, :_NC] = 1.0
    m[1, _NC:] = 1.0
    return jnp.asarray(m)


def _bsq_kernel(z_ref, w_ref, ones_ref, zq_ref, idx_ref, gidx_ref, avgp_ref,
                loss_ref, cbe_ref, acc_ref, s_ref, *, ntot):
    pid = pl.program_id(0)
    nsteps = pl.num_programs(0)

    @pl.when(pid == 0)
    def _init():
        acc_ref[...] = jnp.zeros_like(acc_ref)
        s_ref[0] = jnp.float32(0.0)
        s_ref[1] = jnp.float32(0.0)

    z = z_ref[...]
    zhat = jnp.where(z > 0, jnp.float32(1.0), jnp.float32(-1.0))
    zq = zhat * _QS
    zq_ref[...] = zq

    cz = z * jnp.float32(2.0 * float(_QS))
    hi = cz.astype(jnp.bfloat16).astype(jnp.float32)
    lo = cz - hi
    x = jnp.concatenate([hi, lo, zhat], axis=1)
    o1 = jnp.dot(x, w_ref[...], preferred_element_type=jnp.float32)

    idx_f = 131071.5 + _HALF_QS * o1[:, 2 * _NC:2 * _NC + 1]
    idx_ref[...] = idx_f.astype(jnp.int32)
    g = 255.5 + _HALF_QS * o1[:, 2 * _NC + 1:]
    gidx_ref[...] = g.astype(jnp.int32)

    lg = o1[:, :2 * _NC]
    e = jnp.exp(lg)
    el = e * lg
    s2 = jax.lax.dot_general(ones_ref[...], e, (((1,), (1,)), ((), ())),
                             preferred_element_type=jnp.float32)
    t2 = jax.lax.dot_general(ones_ref[...], el, (((1,), (1,)), ((), ())),
                             preferred_element_type=jnp.float32)
    r2 = 1.0 / s2
    res = jax.lax.dot_general(r2, e, (((1,), (0,)), ((), ())),
                              preferred_element_type=jnp.float32)
    acc_ref[...] += res

    s_ref[0] += jnp.sum(jnp.log(s2) - t2 * r2)
    diff = zq - z
    s_ref[1] += jnp.sum(diff * diff)

    @pl.when(pid == nsteps - 1)
    def _fin():
        inv_n = jnp.float32(1.0 / ntot)
        acc = acc_ref[...] * inv_n
        avgp = jnp.concatenate([acc[0:1, :_NC], acc[1:2, _NC:]], axis=0)
        avgp_ref[...] = avgp
        cbe = -jnp.sum(avgp * jnp.log(avgp + 1e-8))
        cbe_ref[...] = jnp.reshape(cbe, (1, 1))
        pse = s_ref[0] * inv_n
        commit = 0.25 * (s_ref[1] * inv_n)
        loss_ref[...] = jnp.reshape(commit + pse - cbe, (1, 1))


def kernel(z):
    b, s, d = z.shape
    n = b * s
    zf = z.reshape(n, d)
    w = _weights()
    ones_bd = _group_ones()
    grid = n // _ROWS
    outs = pl.pallas_call(
        functools.partial(_bsq_kernel, ntot=float(n)),
        grid=(grid,),
        in_specs=[
            pl.BlockSpec((_ROWS, d), lambda i: (i, 0)),
            pl.BlockSpec((3 * _D, _NOUT), lambda i: (0, 0)),
            pl.BlockSpec((2, 2 * _NC), lambda i: (0, 0)),
        ],
        out_specs=[
            pl.BlockSpec((_ROWS, d), lambda i: (i, 0)),
            pl.BlockSpec((_ROWS, 1), lambda i: (i, 0)),
            pl.BlockSpec((_ROWS, 2), lambda i: (i, 0)),
            pl.BlockSpec((2, _NC), lambda i: (0, 0)),
            pl.BlockSpec((1, 1), lambda i: (0, 0)),
            pl.BlockSpec((1, 1), lambda i: (0, 0)),
        ],
        out_shape=[
            jax.ShapeDtypeStruct((n, d), jnp.float32),
            jax.ShapeDtypeStruct((n, 1), jnp.int32),
            jax.ShapeDtypeStruct((n, 2), jnp.int32),
            jax.ShapeDtypeStruct((2, _NC), jnp.float32),
            jax.ShapeDtypeStruct((1, 1), jnp.float32),
            jax.ShapeDtypeStruct((1, 1), jnp.float32),
        ],
        scratch_shapes=[
            pltpu.VMEM((2, 2 * _NC), jnp.float32),
            pltpu.SMEM((2,), jnp.float32),
        ],
        compiler_params=pltpu.CompilerParams(
            dimension_semantics=("arbitrary",)),
    )(zf, w, ones_bd)
    zq, idx, gidx, avgp, loss, cbe = outs
    zq = zq.reshape(b, s, d)
    indices = idx.reshape(b, s).astype(jnp.int64)
    group_indices = gidx.reshape(b, s, 2).astype(jnp.int64)
    return (zq, loss[0, 0], cbe[0, 0], indices, group_indices, avgp)


# speedup vs baseline: 1.1147x; 1.1147x over previous
import functools

import numpy as np
import jax
import jax.numpy as jnp
from jax.experimental import pallas as pl
from jax.experimental.pallas import tpu as pltpu

_D = 18
_GS = 9
_NA = 16
_NB = 32
_NSMALL = 2 * (_NA + _NB)
_NOUT = _NSMALL + 3
_SQRT_D = np.float32(np.sqrt(np.float32(18.0)))
_QS = np.float32(np.float32(1.0) / _SQRT_D)
_HALF_QS = np.float32(_QS / np.float32(2.0))
_ROWS = 1024


def _pats(nbits):
    codes = np.arange(1 << nbits)
    gb = 2 ** np.arange(nbits - 1, -1, -1)
    return (((codes[:, None] // gb) % 2) * 2 - 1).astype(np.float32)


def _weights():
    cba = _pats(4)
    cbb = _pats(5)
    w = np.zeros((3 * _D, _NOUT), np.float32)
    w[0:4, 0:16] = cba.T
    w[4:9, 16:48] = cbb.T
    w[9:13, 48:64] = cba.T
    w[13:18, 64:96] = cbb.T
    w[_D:2 * _D, :_NSMALL] = w[:_D, :_NSMALL]
    sb = (2.0 ** np.arange(_D - 1 - _D, -1 - _D, -1)).astype(np.float32)
    w[2 * _D:, 96] = sb
    gb = (2.0 ** np.arange(_GS - 1 - _GS, -1 - _GS, -1)).astype(np.float32)
    w[2 * _D:2 * _D + _GS, 97] = gb
    w[2 * _D + _GS:, 98] = gb
    return jnp.asarray(w)


def _sum_pattern():
    p = np.zeros((_NSMALL, 4), np.float32)
    p[0:16, 0] = 1.0
    p[16:48, 1] = 1.0
    p[48:64, 2] = 1.0
    p[64:96, 3] = 1.0
    return jnp.asarray(p)


def _bsq_kernel(z_ref, w_ref, p_ref, zq_ref, idx_ref, gidx_ref, avgp_ref,
                loss_ref, cbe_ref, acc_ref, s_ref, *, ntot):
    pid = pl.program_id(0)
    nsteps = pl.num_programs(0)

    @pl.when(pid == 0)
    def _init():
        acc_ref[...] = jnp.zeros_like(acc_ref)
        s_ref[0] = jnp.float32(0.0)
        s_ref[1] = jnp.float32(0.0)

    z = z_ref[...]
    zhat = jnp.where(z > 0, jnp.float32(1.0), jnp.float32(-1.0))
    zq = zhat * _QS
    zq_ref[...] = zq

    cz = z * jnp.float32(2.0 * float(_QS))
    hi = cz.astype(jnp.bfloat16).astype(jnp.float32)
    lo = cz - hi
    x = jnp.concatenate([hi, lo, zhat], axis=1)
    o1 = jnp.dot(x, w_ref[...], preferred_element_type=jnp.float32)

    idx_f = 131071.5 + (_HALF_QS * 262144.0) * o1[:, _NSMALL:_NSMALL + 1]
    idx_ref[...] = idx_f.astype(jnp.int32)
    g = 255.5 + (_HALF_QS * 512.0) * o1[:, _NSMALL + 1:]
    gidx_ref[...] = g.astype(jnp.int32)

    lg = o1[:, :_NSMALL]
    e = jnp.exp(lg)
    el = e * lg
    s4 = jnp.dot(e, p_ref[...], preferred_element_type=jnp.float32)
    t4 = jnp.dot(el, p_ref[...], preferred_element_type=jnp.float32)
    ra = 1.0 / s4
    s_ref[0] += jnp.sum(jnp.log(s4)) - jnp.sum(t4 * ra)
    r0 = ra[:, 0:1] * ra[:, 1:2]
    r1 = ra[:, 2:3] * ra[:, 3:4]
    a0 = jax.lax.dot_general(e[:, :_NA] * r0, e[:, _NA:_NA + _NB],
                             (((0,), (0,)), ((), ())),
                             preferred_element_type=jnp.float32)
    a1 = jax.lax.dot_general(e[:, 48:48 + _NA] * r1, e[:, 64:],
                             (((0,), (0,)), ((), ())),
                             preferred_element_type=jnp.float32)
    acc_ref[:, :_NB] += a0
    acc_ref[:, _NB:] += a1

    diff = zq - z
    s_ref[1] += jnp.sum(diff * diff)

    @pl.when(pid == nsteps - 1)
    def _fin():
        inv_n = jnp.float32(1.0 / ntot)
        acc = acc_ref[...] * inv_n
        avgp_ref[...] = acc
        cbe = -jnp.sum(acc * jnp.log(acc + 1e-8))
        cbe_ref[...] = jnp.reshape(cbe, (1, 1))
        pse = s_ref[0] * inv_n
        commit = 0.25 * (s_ref[1] * inv_n)
        loss_ref[...] = jnp.reshape(commit + pse - cbe, (1, 1))


def kernel(z):
    b, s, d = z.shape
    n = b * s
    zf = z.reshape(n, d)
    w = _weights()
    pat = _sum_pattern()
    grid = n // _ROWS
    outs = pl.pallas_call(
        functools.partial(_bsq_kernel, ntot=float(n)),
        grid=(grid,),
        in_specs=[
            pl.BlockSpec((_ROWS, d), lambda i: (i, 0)),
            pl.BlockSpec((3 * _D, _NOUT), lambda i: (0, 0)),
            pl.BlockSpec((_NSMALL, 4), lambda i: (0, 0)),
        ],
        out_specs=[
            pl.BlockSpec((_ROWS, d), lambda i: (i, 0)),
            pl.BlockSpec((_ROWS, 1), lambda i: (i, 0)),
            pl.BlockSpec((_ROWS, 2), lambda i: (i, 0)),
            pl.BlockSpec((_NA, 2 * _NB), lambda i: (0, 0)),
            pl.BlockSpec((1, 1), lambda i: (0, 0)),
            pl.BlockSpec((1, 1), lambda i: (0, 0)),
        ],
        out_shape=[
            jax.ShapeDtypeStruct((n, d), jnp.float32),
            jax.ShapeDtypeStruct((n, 1), jnp.int32),
            jax.ShapeDtypeStruct((n, 2), jnp.int32),
            jax.ShapeDtypeStruct((_NA, 2 * _NB), jnp.float32),
            jax.ShapeDtypeStruct((1, 1), jnp.float32),
            jax.ShapeDtypeStruct((1, 1), jnp.float32),
        ],
        scratch_shapes=[
            pltpu.VMEM((_NA, 2 * _NB), jnp.float32),
            pltpu.SMEM((2,), jnp.float32),
        ],
        compiler_params=pltpu.CompilerParams(
            dimension_semantics=("arbitrary",)),
    )(zf, w, pat)
    zq, idx, gidx, avgp_raw, loss, cbe = outs
    zq = zq.reshape(b, s, d)
    indices = idx.reshape(b, s).astype(jnp.int64)
    group_indices = gidx.reshape(b, s, 2).astype(jnp.int64)
    avgp = jnp.stack([avgp_raw[:, :_NB].reshape(_NA * _NB),
                      avgp_raw[:, _NB:].reshape(_NA * _NB)], axis=0)
    return (zq, loss[0, 0], cbe[0, 0], indices, group_indices, avgp)


# speedup vs baseline: 1.8922x; 1.6975x over previous
import functools

import numpy as np
import jax
import jax.numpy as jnp
from jax.experimental import pallas as pl
from jax.experimental.pallas import tpu as pltpu

_D = 18
_GS = 9
_NA = 16
_NB = 32
_NSMALL = 2 * (_NA + _NB)
_NOUT = _NSMALL + 3
_SQRT_D = np.float32(np.sqrt(np.float32(18.0)))
_QS = np.float32(np.float32(1.0) / _SQRT_D)
_HALF_QS = np.float32(_QS / np.float32(2.0))
_ROWS = 1024


def _pats(nbits):
    codes = np.arange(1 << nbits)
    gb = 2 ** np.arange(nbits - 1, -1, -1)
    return (((codes[:, None] // gb) % 2) * 2 - 1).astype(np.float32)


def _weights_t():
    cba = _pats(4)
    cbb = _pats(5)
    w = np.zeros((3 * _D, _NOUT), np.float32)
    w[0:4, 0:16] = cba.T
    w[4:9, 16:48] = cbb.T
    w[9:13, 48:64] = cba.T
    w[13:18, 64:96] = cbb.T
    w[_D:2 * _D, :_NSMALL] = w[:_D, :_NSMALL]
    sb = (2.0 ** np.arange(-1, -1 - _D, -1)).astype(np.float32)
    w[2 * _D:, 96] = sb
    gb = (2.0 ** np.arange(-1, -1 - _GS, -1)).astype(np.float32)
    w[2 * _D:2 * _D + _GS, 97] = gb
    w[2 * _D + _GS:, 98] = gb
    return jnp.asarray(w.T.copy())


def _sum_pattern_t():
    p = np.zeros((4, _NSMALL), np.float32)
    p[0, 0:16] = 1.0
    p[1, 16:48] = 1.0
    p[2, 48:64] = 1.0
    p[3, 64:96] = 1.0
    return jnp.asarray(p)


def _bsq_kernel(z_ref, w_ref, p_ref, zq_ref, idx3_ref, avgp_ref,
                loss_ref, cbe_ref, acc_ref, s_ref, *, ntot):
    pid = pl.program_id(0)
    nsteps = pl.num_programs(0)

    @pl.when(pid == 0)
    def _init():
        acc_ref[...] = jnp.zeros_like(acc_ref)
        s_ref[0] = jnp.float32(0.0)
        s_ref[1] = jnp.float32(0.0)

    z = z_ref[...]
    zhat = jnp.where(z > 0, jnp.float32(1.0), jnp.float32(-1.0))
    zq = zhat * _QS
    zq_ref[...] = zq

    cz = z * jnp.float32(2.0 * float(_QS))
    hi = cz.astype(jnp.bfloat16).astype(jnp.float32)
    lo = cz - hi
    x = jnp.concatenate([hi, lo, zhat], axis=1)
    o1t = jax.lax.dot_general(w_ref[...], x, (((1,), (1,)), ((), ())),
                              preferred_element_type=jnp.float32)

    idx_f = 131071.5 + (_HALF_QS * 262144.0) * o1t[96:97, :]
    g = 255.5 + (_HALF_QS * 512.0) * o1t[97:99, :]
    idx3_ref[...] = jnp.concatenate([idx_f, g], axis=0).astype(jnp.int32)

    lgt = o1t[:_NSMALL, :]
    et = jnp.exp(lgt)
    elt = et * lgt
    s4 = jax.lax.dot_general(p_ref[...], et, (((1,), (0,)), ((), ())),
                             preferred_element_type=jnp.float32)
    t4 = jax.lax.dot_general(p_ref[...], elt, (((1,), (0,)), ((), ())),
                             preferred_element_type=jnp.float32)
    ra = 1.0 / s4
    s_ref[0] += jnp.sum(jnp.log(s4)) - jnp.sum(t4 * ra)
    r0 = ra[0:1, :] * ra[1:2, :]
    r1 = ra[2:3, :] * ra[3:4, :]
    a0 = jax.lax.dot_general(et[:_NA, :] * r0, et[_NA:_NA + _NB, :],
                             (((1,), (1,)), ((), ())),
                             preferred_element_type=jnp.float32)
    a1 = jax.lax.dot_general(et[48:48 + _NA, :] * r1, et[64:, :],
                             (((1,), (1,)), ((), ())),
                             preferred_element_type=jnp.float32)
    acc_ref[:, :_NB] += a0
    acc_ref[:, _NB:] += a1

    diff = zq - z
    s_ref[1] += jnp.sum(diff * diff)

    @pl.when(pid == nsteps - 1)
    def _fin():
        inv_n = jnp.float32(1.0 / ntot)
        acc = acc_ref[...] * inv_n
        avgp_ref[...] = acc
        cbe = -jnp.sum(acc * jnp.log(acc + 1e-8))
        cbe_ref[...] = jnp.reshape(cbe, (1, 1))
        pse = s_ref[0] * inv_n
        commit = 0.25 * (s_ref[1] * inv_n)
        loss_ref[...] = jnp.reshape(commit + pse - cbe, (1, 1))


def kernel(z):
    b, s, d = z.shape
    n = b * s
    zf = z.reshape(n, d)
    wt = _weights_t()
    pat = _sum_pattern_t()
    grid = n // _ROWS
    outs = pl.pallas_call(
        functools.partial(_bsq_kernel, ntot=float(n)),
        grid=(grid,),
        in_specs=[
            pl.BlockSpec((_ROWS, d), lambda i: (i, 0)),
            pl.BlockSpec((_NOUT, 3 * _D), lambda i: (0, 0)),
            pl.BlockSpec((4, _NSMALL), lambda i: (0, 0)),
        ],
        out_specs=[
            pl.BlockSpec((_ROWS, d), lambda i: (i, 0)),
            pl.BlockSpec((3, _ROWS), lambda i: (0, i)),
            pl.BlockSpec((_NA, 2 * _NB), lambda i: (0, 0)),
            pl.BlockSpec((1, 1), lambda i: (0, 0)),
            pl.BlockSpec((1, 1), lambda i: (0, 0)),
        ],
        out_shape=[
            jax.ShapeDtypeStruct((n, d), jnp.float32),
            jax.ShapeDtypeStruct((3, n), jnp.int32),
            jax.ShapeDtypeStruct((_NA, 2 * _NB), jnp.float32),
            jax.ShapeDtypeStruct((1, 1), jnp.float32),
            jax.ShapeDtypeStruct((1, 1), jnp.float32),
        ],
        scratch_shapes=[
            pltpu.VMEM((_NA, 2 * _NB), jnp.float32),
            pltpu.SMEM((2,), jnp.float32),
        ],
        compiler_params=pltpu.CompilerParams(
            dimension_semantics=("arbitrary",)),
    )(zf, wt, pat)
    zq, idx3, avgp_raw, loss, cbe = outs
    zq = zq.reshape(b, s, d)
    indices = idx3[0].reshape(b, s).astype(jnp.int64)
    group_indices = idx3[1:3].T.reshape(b, s, 2).astype(jnp.int64)
    avgp = jnp.stack([avgp_raw[:, :_NB].reshape(_NA * _NB),
                      avgp_raw[:, _NB:].reshape(_NA * _NB)], axis=0)
    return (zq, loss[0, 0], cbe[0, 0], indices, group_indices, avgp)


# speedup vs baseline: 2.2948x; 1.2128x over previous
import functools

import numpy as np
import jax
import jax.numpy as jnp
from jax.experimental import pallas as pl
from jax.experimental.pallas import tpu as pltpu

_D = 18
_GS = 9
_NA = 16
_NB = 32
_NSMALL = 2 * (_NA + _NB)
_NOUT = _NSMALL + 3
_SQRT_D = np.float32(np.sqrt(np.float32(18.0)))
_QS = np.float32(np.float32(1.0) / _SQRT_D)
_HALF_QS = np.float32(_QS / np.float32(2.0))
_ROWS = 2048


def _pats(nbits):
    codes = np.arange(1 << nbits)
    gb = 2 ** np.arange(nbits - 1, -1, -1)
    return (((codes[:, None] // gb) % 2) * 2 - 1).astype(np.float32)


def _weights_t():
    cba = _pats(4)
    cbb = _pats(5)
    w = np.zeros((3 * _D, _NOUT), np.float32)
    w[0:4, 0:16] = cba.T
    w[4:9, 16:48] = cbb.T
    w[9:13, 48:64] = cba.T
    w[13:18, 64:96] = cbb.T
    w[_D:2 * _D, :_NSMALL] = w[:_D, :_NSMALL]
    sb = (2.0 ** np.arange(-1, -1 - _D, -1)).astype(np.float32)
    w[2 * _D:, 96] = sb
    gb = (2.0 ** np.arange(-1, -1 - _GS, -1)).astype(np.float32)
    w[2 * _D:2 * _D + _GS, 97] = gb
    w[2 * _D + _GS:, 98] = gb
    return jnp.asarray(w.T.copy())


def _sum_pattern_t():
    p = np.zeros((4, _NSMALL), np.float32)
    p[0, 0:16] = 1.0
    p[1, 16:48] = 1.0
    p[2, 48:64] = 1.0
    p[3, 64:96] = 1.0
    return jnp.asarray(p)


def _bsq_kernel(z_ref, w_ref, p_ref, zq_ref, idx3_ref, avgp_ref,
                loss_ref, cbe_ref, acc_ref, s_ref, *, ntot):
    pid = pl.program_id(0)
    nsteps = pl.num_programs(0)

    @pl.when(pid == 0)
    def _init():
        acc_ref[...] = jnp.zeros_like(acc_ref)
        s_ref[0] = jnp.float32(0.0)
        s_ref[1] = jnp.float32(0.0)

    z = z_ref[...]
    zhat = jnp.where(z > 0, jnp.float32(1.0), jnp.float32(-1.0))
    zq = zhat * _QS
    zq_ref[...] = zq

    cz = z * jnp.float32(2.0 * float(_QS))
    hi = cz.astype(jnp.bfloat16).astype(jnp.float32)
    lo = cz - hi
    x = jnp.concatenate([hi, lo, zhat], axis=1)
    o1t = jax.lax.dot_general(w_ref[...], x, (((1,), (1,)), ((), ())),
                              preferred_element_type=jnp.float32)

    idx_f = 131071.5 + (_HALF_QS * 262144.0) * o1t[96:97, :]
    g = 255.5 + (_HALF_QS * 512.0) * o1t[97:99, :]
    idx3_ref[...] = jnp.concatenate([idx_f, g], axis=0).astype(jnp.int32)

    lgt = o1t[:_NSMALL, :]
    et = jnp.exp(lgt)
    elt = et * lgt
    s4 = jax.lax.dot_general(p_ref[...], et, (((1,), (0,)), ((), ())),
                             preferred_element_type=jnp.float32)
    t4 = jax.lax.dot_general(p_ref[...], elt, (((1,), (0,)), ((), ())),
                             preferred_element_type=jnp.float32)
    ra = 1.0 / s4
    s_ref[0] += jnp.sum(jnp.log(s4)) - jnp.sum(t4 * ra)
    r0 = ra[0:1, :] * ra[1:2, :]
    r1 = ra[2:3, :] * ra[3:4, :]
    a0 = jax.lax.dot_general(et[:_NA, :] * r0, et[_NA:_NA + _NB, :],
                             (((1,), (1,)), ((), ())),
                             preferred_element_type=jnp.float32)
    a1 = jax.lax.dot_general(et[48:48 + _NA, :] * r1, et[64:, :],
                             (((1,), (1,)), ((), ())),
                             preferred_element_type=jnp.float32)
    acc_ref[:, :_NB] += a0
    acc_ref[:, _NB:] += a1

    diff = zq - z
    s_ref[1] += jnp.sum(diff * diff)

    @pl.when(pid == nsteps - 1)
    def _fin():
        inv_n = jnp.float32(1.0 / ntot)
        acc = acc_ref[...] * inv_n
        avgp_ref[...] = acc
        cbe = -jnp.sum(acc * jnp.log(acc + 1e-8))
        cbe_ref[...] = jnp.reshape(cbe, (1, 1))
        pse = s_ref[0] * inv_n
        commit = 0.25 * (s_ref[1] * inv_n)
        loss_ref[...] = jnp.reshape(commit + pse - cbe, (1, 1))


def kernel(z):
    b, s, d = z.shape
    n = b * s
    zf = z.reshape(n, d)
    wt = _weights_t()
    pat = _sum_pattern_t()
    grid = n // _ROWS
    outs = pl.pallas_call(
        functools.partial(_bsq_kernel, ntot=float(n)),
        grid=(grid,),
        in_specs=[
            pl.BlockSpec((_ROWS, d), lambda i: (i, 0)),
            pl.BlockSpec((_NOUT, 3 * _D), lambda i: (0, 0)),
            pl.BlockSpec((4, _NSMALL), lambda i: (0, 0)),
        ],
        out_specs=[
            pl.BlockSpec((_ROWS, d), lambda i: (i, 0)),
            pl.BlockSpec((3, _ROWS), lambda i: (0, i)),
            pl.BlockSpec((_NA, 2 * _NB), lambda i: (0, 0)),
            pl.BlockSpec((1, 1), lambda i: (0, 0)),
            pl.BlockSpec((1, 1), lambda i: (0, 0)),
        ],
        out_shape=[
            jax.ShapeDtypeStruct((n, d), jnp.float32),
            jax.ShapeDtypeStruct((3, n), jnp.int32),
            jax.ShapeDtypeStruct((_NA, 2 * _NB), jnp.float32),
            jax.ShapeDtypeStruct((1, 1), jnp.float32),
            jax.ShapeDtypeStruct((1, 1), jnp.float32),
        ],
        scratch_shapes=[
            pltpu.VMEM((_NA, 2 * _NB), jnp.float32),
            pltpu.SMEM((2,), jnp.float32),
        ],
        compiler_params=pltpu.CompilerParams(
            dimension_semantics=("arbitrary",)),
    )(zf, wt, pat)
    zq, idx3, avgp_raw, loss, cbe = outs
    zq = zq.reshape(b, s, d)
    indices = idx3[0].reshape(b, s).astype(jnp.int64)
    group_indices = idx3[1:3].T.reshape(b, s, 2).astype(jnp.int64)
    avgp = jnp.stack([avgp_raw[:, :_NB].reshape(_NA * _NB),
                      avgp_raw[:, _NB:].reshape(_NA * _NB)], axis=0)
    return (zq, loss[0, 0], cbe[0, 0], indices, group_indices, avgp)


# speedup vs baseline: 2.5495x; 1.1110x over previous
import functools

import numpy as np
import jax
import jax.numpy as jnp
from jax.experimental import pallas as pl
from jax.experimental.pallas import tpu as pltpu

_D = 18
_GS = 9
_NA = 16
_NB = 32
_NSMALL = 2 * (_NA + _NB)
_NOUT = _NSMALL + 3
_SQRT_D = np.float32(np.sqrt(np.float32(18.0)))
_QS = np.float32(np.float32(1.0) / _SQRT_D)
_HALF_QS = np.float32(_QS / np.float32(2.0))
_ROWS = 4096


def _pats(nbits):
    codes = np.arange(1 << nbits)
    gb = 2 ** np.arange(nbits - 1, -1, -1)
    return (((codes[:, None] // gb) % 2) * 2 - 1).astype(np.float32)


def _weights_t():
    cba = _pats(4)
    cbb = _pats(5)
    w = np.zeros((3 * _D, _NOUT), np.float32)
    w[0:4, 0:16] = cba.T
    w[4:9, 16:48] = cbb.T
    w[9:13, 48:64] = cba.T
    w[13:18, 64:96] = cbb.T
    w[_D:2 * _D, :_NSMALL] = w[:_D, :_NSMALL]
    sb = (2.0 ** np.arange(-1, -1 - _D, -1)).astype(np.float32)
    w[2 * _D:, 96] = sb
    gb = (2.0 ** np.arange(-1, -1 - _GS, -1)).astype(np.float32)
    w[2 * _D:2 * _D + _GS, 97] = gb
    w[2 * _D + _GS:, 98] = gb
    return jnp.asarray(w.T.copy())


def _sum_pattern_t():
    p = np.zeros((4, _NSMALL), np.float32)
    p[0, 0:16] = 1.0
    p[1, 16:48] = 1.0
    p[2, 48:64] = 1.0
    p[3, 64:96] = 1.0
    return jnp.asarray(p)


def _bsq_kernel(z_ref, w_ref, p_ref, zq_ref, idx3_ref, avgp_ref,
                loss_ref, cbe_ref, acc_ref, s_ref, *, ntot):
    pid = pl.program_id(0)
    nsteps = pl.num_programs(0)

    @pl.when(pid == 0)
    def _init():
        acc_ref[...] = jnp.zeros_like(acc_ref)
        s_ref[0] = jnp.float32(0.0)
        s_ref[1] = jnp.float32(0.0)

    z = z_ref[...]
    zhat = jnp.where(z > 0, jnp.float32(1.0), jnp.float32(-1.0))
    zq = zhat * _QS
    zq_ref[...] = zq

    cz = z * jnp.float32(2.0 * float(_QS))
    hi = cz.astype(jnp.bfloat16).astype(jnp.float32)
    lo = cz - hi
    x = jnp.concatenate([hi, lo, zhat], axis=1)
    o1t = jax.lax.dot_general(w_ref[...], x, (((1,), (1,)), ((), ())),
                              preferred_element_type=jnp.float32)

    idx_f = 131071.5 + (_HALF_QS * 262144.0) * o1t[96:97, :]
    g = 255.5 + (_HALF_QS * 512.0) * o1t[97:99, :]
    idx3_ref[...] = jnp.concatenate([idx_f, g], axis=0).astype(jnp.int32)

    lgt = o1t[:_NSMALL, :]
    et = jnp.exp(lgt)
    elt = et * lgt
    s4 = jax.lax.dot_general(p_ref[...], et, (((1,), (0,)), ((), ())),
                             preferred_element_type=jnp.float32)
    t4 = jax.lax.dot_general(p_ref[...], elt, (((1,), (0,)), ((), ())),
                             preferred_element_type=jnp.float32)
    ra = 1.0 / s4
    s_ref[0] += jnp.sum(jnp.log(s4)) - jnp.sum(t4 * ra)
    r0 = ra[0:1, :] * ra[1:2, :]
    r1 = ra[2:3, :] * ra[3:4, :]
    a0 = jax.lax.dot_general(et[:_NA, :] * r0, et[_NA:_NA + _NB, :],
                             (((1,), (1,)), ((), ())),
                             preferred_element_type=jnp.float32)
    a1 = jax.lax.dot_general(et[48:48 + _NA, :] * r1, et[64:, :],
                             (((1,), (1,)), ((), ())),
                             preferred_element_type=jnp.float32)
    acc_ref[:, :_NB] += a0
    acc_ref[:, _NB:] += a1

    diff = zq - z
    s_ref[1] += jnp.sum(diff * diff)

    @pl.when(pid == nsteps - 1)
    def _fin():
        inv_n = jnp.float32(1.0 / ntot)
        acc = acc_ref[...] * inv_n
        avgp_ref[...] = acc
        cbe = -jnp.sum(acc * jnp.log(acc + 1e-8))
        cbe_ref[...] = jnp.reshape(cbe, (1, 1))
        pse = s_ref[0] * inv_n
        commit = 0.25 * (s_ref[1] * inv_n)
        loss_ref[...] = jnp.reshape(commit + pse - cbe, (1, 1))


def kernel(z):
    b, s, d = z.shape
    n = b * s
    zf = z.reshape(n, d)
    wt = _weights_t()
    pat = _sum_pattern_t()
    grid = n // _ROWS
    outs = pl.pallas_call(
        functools.partial(_bsq_kernel, ntot=float(n)),
        grid=(grid,),
        in_specs=[
            pl.BlockSpec((_ROWS, d), lambda i: (i, 0)),
            pl.BlockSpec((_NOUT, 3 * _D), lambda i: (0, 0)),
            pl.BlockSpec((4, _NSMALL), lambda i: (0, 0)),
        ],
        out_specs=[
            pl.BlockSpec((_ROWS, d), lambda i: (i, 0)),
            pl.BlockSpec((3, _ROWS), lambda i: (0, i)),
            pl.BlockSpec((_NA, 2 * _NB), lambda i: (0, 0)),
            pl.BlockSpec((1, 1), lambda i: (0, 0)),
            pl.BlockSpec((1, 1), lambda i: (0, 0)),
        ],
        out_shape=[
            jax.ShapeDtypeStruct((n, d), jnp.float32),
            jax.ShapeDtypeStruct((3, n), jnp.int32),
            jax.ShapeDtypeStruct((_NA, 2 * _NB), jnp.float32),
            jax.ShapeDtypeStruct((1, 1), jnp.float32),
            jax.ShapeDtypeStruct((1, 1), jnp.float32),
        ],
        scratch_shapes=[
            pltpu.VMEM((_NA, 2 * _NB), jnp.float32),
            pltpu.SMEM((2,), jnp.float32),
        ],
        compiler_params=pltpu.CompilerParams(
            dimension_semantics=("arbitrary",)),
    )(zf, wt, pat)
    zq, idx3, avgp_raw, loss, cbe = outs
    zq = zq.reshape(b, s, d)
    indices = idx3[0].reshape(b, s).astype(jnp.int64)
    group_indices = idx3[1:3].T.reshape(b, s, 2).astype(jnp.int64)
    avgp = jnp.stack([avgp_raw[:, :_NB].reshape(_NA * _NB),
                      avgp_raw[:, _NB:].reshape(_NA * _NB)], axis=0)
    return (zq, loss[0, 0], cbe[0, 0], indices, group_indices, avgp)


# speedup vs baseline: 2.5988x; 1.0193x over previous
import functools

import numpy as np
import jax
import jax.numpy as jnp
from jax.experimental import pallas as pl
from jax.experimental.pallas import tpu as pltpu

_D = 18
_GS = 9
_NA = 16
_NB = 32
_NSMALL = 2 * (_NA + _NB)
_NOUT = _NSMALL + 3
_SQRT_D = np.float32(np.sqrt(np.float32(18.0)))
_QS = np.float32(np.float32(1.0) / _SQRT_D)
_HALF_QS = np.float32(_QS / np.float32(2.0))
_ROWS = 8192


def _pats(nbits):
    codes = np.arange(1 << nbits)
    gb = 2 ** np.arange(nbits - 1, -1, -1)
    return (((codes[:, None] // gb) % 2) * 2 - 1).astype(np.float32)


def _weights_t():
    cba = _pats(4)
    cbb = _pats(5)
    w = np.zeros((3 * _D, _NOUT), np.float32)
    w[0:4, 0:16] = cba.T
    w[4:9, 16:48] = cbb.T
    w[9:13, 48:64] = cba.T
    w[13:18, 64:96] = cbb.T
    w[_D:2 * _D, :_NSMALL] = w[:_D, :_NSMALL]
    sb = (2.0 ** np.arange(-1, -1 - _D, -1)).astype(np.float32)
    w[2 * _D:, 96] = sb
    gb = (2.0 ** np.arange(-1, -1 - _GS, -1)).astype(np.float32)
    w[2 * _D:2 * _D + _GS, 97] = gb
    w[2 * _D + _GS:, 98] = gb
    return jnp.asarray(w.T.copy())


def _sum_pattern_t():
    p = np.zeros((4, _NSMALL), np.float32)
    p[0, 0:16] = 1.0
    p[1, 16:48] = 1.0
    p[2, 48:64] = 1.0
    p[3, 64:96] = 1.0
    return jnp.asarray(p)


def _bsq_kernel(z_ref, w_ref, p_ref, zq_ref, idx3_ref, avgp_ref,
                loss_ref, cbe_ref, acc_ref, s_ref, *, ntot):
    pid = pl.program_id(0)
    nsteps = pl.num_programs(0)

    @pl.when(pid == 0)
    def _init():
        acc_ref[...] = jnp.zeros_like(acc_ref)
        s_ref[0] = jnp.float32(0.0)
        s_ref[1] = jnp.float32(0.0)

    z = z_ref[...]
    zhat = jnp.where(z > 0, jnp.float32(1.0), jnp.float32(-1.0))
    zq = zhat * _QS
    zq_ref[...] = zq

    cz = z * jnp.float32(2.0 * float(_QS))
    hi = cz.astype(jnp.bfloat16).astype(jnp.float32)
    lo = cz - hi
    x = jnp.concatenate([hi, lo, zhat], axis=1)
    o1t = jax.lax.dot_general(w_ref[...], x, (((1,), (1,)), ((), ())),
                              preferred_element_type=jnp.float32)

    idx_f = 131071.5 + (_HALF_QS * 262144.0) * o1t[96:97, :]
    g = 255.5 + (_HALF_QS * 512.0) * o1t[97:99, :]
    idx3_ref[...] = jnp.concatenate([idx_f, g], axis=0).astype(jnp.int32)

    lgt = o1t[:_NSMALL, :]
    et = jnp.exp(lgt)
    elt = et * lgt
    s4 = jax.lax.dot_general(p_ref[...], et, (((1,), (0,)), ((), ())),
                             preferred_element_type=jnp.float32)
    t4 = jax.lax.dot_general(p_ref[...], elt, (((1,), (0,)), ((), ())),
                             preferred_element_type=jnp.float32)
    ra = 1.0 / s4
    s_ref[0] += jnp.sum(jnp.log(s4)) - jnp.sum(t4 * ra)
    r0 = ra[0:1, :] * ra[1:2, :]
    r1 = ra[2:3, :] * ra[3:4, :]
    a0 = jax.lax.dot_general(et[:_NA, :] * r0, et[_NA:_NA + _NB, :],
                             (((1,), (1,)), ((), ())),
                             preferred_element_type=jnp.float32)
    a1 = jax.lax.dot_general(et[48:48 + _NA, :] * r1, et[64:, :],
                             (((1,), (1,)), ((), ())),
                             preferred_element_type=jnp.float32)
    acc_ref[:, :_NB] += a0
    acc_ref[:, _NB:] += a1

    diff = zq - z
    s_ref[1] += jnp.sum(diff * diff)

    @pl.when(pid == nsteps - 1)
    def _fin():
        inv_n = jnp.float32(1.0 / ntot)
        acc = acc_ref[...] * inv_n
        avgp_ref[...] = acc
        cbe = -jnp.sum(acc * jnp.log(acc + 1e-8))
        cbe_ref[...] = jnp.reshape(cbe, (1, 1))
        pse = s_ref[0] * inv_n
        commit = 0.25 * (s_ref[1] * inv_n)
        loss_ref[...] = jnp.reshape(commit + pse - cbe, (1, 1))


def kernel(z):
    b, s, d = z.shape
    n = b * s
    zf = z.reshape(n, d)
    wt = _weights_t()
    pat = _sum_pattern_t()
    grid = n // _ROWS
    outs = pl.pallas_call(
        functools.partial(_bsq_kernel, ntot=float(n)),
        grid=(grid,),
        in_specs=[
            pl.BlockSpec((_ROWS, d), lambda i: (i, 0)),
            pl.BlockSpec((_NOUT, 3 * _D), lambda i: (0, 0)),
            pl.BlockSpec((4, _NSMALL), lambda i: (0, 0)),
        ],
        out_specs=[
            pl.BlockSpec((_ROWS, d), lambda i: (i, 0)),
            pl.BlockSpec((3, _ROWS), lambda i: (0, i)),
            pl.BlockSpec((_NA, 2 * _NB), lambda i: (0, 0)),
            pl.BlockSpec((1, 1), lambda i: (0, 0)),
            pl.BlockSpec((1, 1), lambda i: (0, 0)),
        ],
        out_shape=[
            jax.ShapeDtypeStruct((n, d), jnp.float32),
            jax.ShapeDtypeStruct((3, n), jnp.int32),
            jax.ShapeDtypeStruct((_NA, 2 * _NB), jnp.float32),
            jax.ShapeDtypeStruct((1, 1), jnp.float32),
            jax.ShapeDtypeStruct((1, 1), jnp.float32),
        ],
        scratch_shapes=[
            pltpu.VMEM((_NA, 2 * _NB), jnp.float32),
            pltpu.SMEM((2,), jnp.float32),
        ],
        compiler_params=pltpu.CompilerParams(
            dimension_semantics=("arbitrary",)),
    )(zf, wt, pat)
    zq, idx3, avgp_raw, loss, cbe = outs
    zq = zq.reshape(b, s, d)
    indices = idx3[0].reshape(b, s).astype(jnp.int64)
    group_indices = idx3[1:3].T.reshape(b, s, 2).astype(jnp.int64)
    avgp = jnp.stack([avgp_raw[:, :_NB].reshape(_NA * _NB),
                      avgp_raw[:, _NB:].reshape(_NA * _NB)], axis=0)
    return (zq, loss[0, 0], cbe[0, 0], indices, group_indices, avgp)
